# baseline (speedup 1.0000x reference)
import functools

import jax
import jax.numpy as jnp
from jax import lax
from jax.experimental import pallas as pl
from jax.experimental.pallas import tpu as pltpu
from jax.experimental.pallas import tpu_sc as plsc

D = 64
BB = 128
NC = 2
NS = 16
NW = NC * NS


def _emb_lookup(idx_t, table_wide):
    hist, batch = idx_t.shape
    nblk = batch // BB
    n_units = hist * nblk
    upw = n_units // NW

    mesh = plsc.VectorSubcoreMesh(core_axis_name="c", subcore_axis_name="s")

    @functools.partial(
        pl.kernel,
        mesh=mesh,
        out_type=jax.ShapeDtypeStruct((hist, D, batch), jnp.float32),
        compiler_params=pltpu.CompilerParams(needs_layout_passes=False),
        scratch_types=[
            pltpu.VMEM((BB,), jnp.int32),
            pltpu.VMEM((BB,), jnp.int32),
            pltpu.VMEM((BB,), jnp.int32),
            pltpu.VMEM((BB, 128), jnp.float32),
            pltpu.VMEM((D, BB), jnp.float32),
            pltpu.SemaphoreType.DMA,
        ],
    )
    def k(idx_hbm, table_hbm, out_hbm, iv, wv, pv, buf, oblk, sem):
        wid = lax.axis_index("s") * NC + lax.axis_index("c")
        lane = lax.iota(jnp.int32, 16)

        def body(t, carry):
            u = wid * upw + t
            h = u // nblk
            blk = u % nblk
            pltpu.sync_copy(idx_hbm.at[h, pl.ds(blk * BB, BB)], iv)
            for g in range(BB // 16):
                v = iv[pl.ds(g * 16, 16)]
                wv[pl.ds(g * 16, 16)] = jax.lax.shift_right_logical(v, 1)
                pv[pl.ds(g * 16, 16)] = jax.lax.shift_left(
                    jax.lax.bitwise_and(v, 1), 6)
            pltpu.async_copy(table_hbm.at[wv], buf, sem).wait()
            for g in range(BB // 16):
                rows = g * 16 + lane
                colbase = pv[pl.ds(g * 16, 16)]
                for j in range(D):
                    vals = plsc.load_gather(buf, [rows, colbase + j])
                    oblk[j, pl.ds(g * 16, 16)] = vals
            pltpu.sync_copy(oblk, out_hbm.at[h, :, pl.ds(blk * BB, BB)])
            return carry

        lax.fori_loop(0, upw, body, 0)

    return k(idx_t, table_wide)


def kernel(itemseq_input, embedding_weight):
    batch, hist = itemseq_input.shape
    idx_t = itemseq_input.astype(jnp.int32).T
    table_wide = embedding_weight.reshape(-1, 2 * D)
    out_t = _emb_lookup(idx_t, table_wide)
    return out_t.transpose(2, 0, 1)


# speedup vs baseline: 1.2305x; 1.2305x over previous
import functools

import jax
import jax.numpy as jnp
from jax import lax
from jax.experimental import pallas as pl
from jax.experimental.pallas import tpu as pltpu
from jax.experimental.pallas import tpu_sc as plsc

D = 64
BB = 128
NC = 2
NS = 16
NW = NC * NS


def _emb_lookup(idx_t, table_wide):
    hist, batch = idx_t.shape
    nblk = batch // BB
    n_units = hist * nblk
    upw = n_units // NW

    mesh = plsc.VectorSubcoreMesh(core_axis_name="c", subcore_axis_name="s")

    @functools.partial(
        pl.kernel,
        mesh=mesh,
        out_type=jax.ShapeDtypeStruct((hist, D, batch), jnp.float32),
        compiler_params=pltpu.CompilerParams(needs_layout_passes=False),
        scratch_types=[
            pltpu.VMEM((2, BB), jnp.int32),
            pltpu.VMEM((2, BB), jnp.int32),
            pltpu.VMEM((2, BB), jnp.int32),
            pltpu.VMEM((2, BB, 128), jnp.float32),
            pltpu.VMEM((2, D, BB), jnp.float32),
            pltpu.SemaphoreType.DMA,
            pltpu.SemaphoreType.DMA,
            pltpu.SemaphoreType.DMA,
        ],
    )
    def k(idx_hbm, table_hbm, out_hbm, iv, wv, pv, buf, oblk, isem, gsem, wsem):
        wid = lax.axis_index("s") * NC + lax.axis_index("c")
        lane = lax.iota(jnp.int32, 16)
        ubase = wid * upw

        def stage_idx(t, sem_async):
            u = ubase + t
            h = u // nblk
            blk = u % nblk
            src = idx_hbm.at[h, pl.ds(blk * BB, BB)]
            dst = iv.at[t % 2]
            if sem_async:
                pltpu.async_copy(src, dst, isem)
            else:
                pltpu.sync_copy(src, dst)

        def wait_idx():
            pltpu.make_async_copy(idx_hbm.at[0, pl.ds(0, BB)], iv.at[0], isem).wait()

        def compute_widx(t):
            s = t % 2
            for g in range(BB // 16):
                v = iv[s, pl.ds(g * 16, 16)]
                wv[s, pl.ds(g * 16, 16)] = lax.shift_right_logical(v, 1)
                pv[s, pl.ds(g * 16, 16)] = lax.shift_left(lax.bitwise_and(v, 1), 6)

        def fire_gather(t):
            pltpu.async_copy(table_hbm.at[wv.at[t % 2]], buf.at[t % 2], gsem)

        def wait_gather():
            pltpu.make_async_copy(
                table_hbm.at[pl.ds(0, BB)], buf.at[0], gsem).wait()

        def select(t):
            s = t % 2
            for g in range(BB // 16):
                rows = g * 16 + lane
                cvec = pv[s, pl.ds(g * 16, 16)]
                for j in range(D):
                    vals = plsc.load_gather(buf.at[s], [rows, cvec + j])
                    oblk[s, j, pl.ds(g * 16, 16)] = vals

        def fire_write(t):
            u = ubase + t
            h = u // nblk
            blk = u % nblk
            pltpu.async_copy(
                oblk.at[t % 2], out_hbm.at[h, :, pl.ds(blk * BB, BB)], wsem)

        def wait_write():
            pltpu.make_async_copy(
                oblk.at[0], out_hbm.at[0, :, pl.ds(0, BB)], wsem).wait()

        stage_idx(0, False)
        compute_widx(0)
        fire_gather(0)
        stage_idx(1, True)

        def body(t, carry):
            @pl.when(t + 1 < upw)
            def _():
                wait_idx()
                compute_widx(t + 1)
                fire_gather(t + 1)

            @pl.when(t + 2 < upw)
            def _():
                stage_idx(t + 2, True)

            wait_gather()

            @pl.when(t >= 2)
            def _():
                wait_write()

            select(t)
            fire_write(t)
            return carry

        lax.fori_loop(0, upw, body, 0)
        wait_write()
        wait_write()

    return k(idx_t, table_wide)


def kernel(itemseq_input, embedding_weight):
    batch, hist = itemseq_input.shape
    idx_t = itemseq_input.astype(jnp.int32).T
    table_wide = embedding_weight.reshape(-1, 2 * D)
    out_t = _emb_lookup(idx_t, table_wide)
    return out_t.transpose(2, 0, 1)


# speedup vs baseline: 1.3199x; 1.0727x over previous
import functools

import jax
import jax.numpy as jnp
from jax import lax
from jax.experimental import pallas as pl
from jax.experimental.pallas import tpu as pltpu
from jax.experimental.pallas import tpu_sc as plsc

D = 64
BB = 128
NC = 2
NS = 16
NW = NC * NS


def _emb_lookup(idx_t, table_wide):
    hist, batch = idx_t.shape
    nblk = batch // BB
    n_units = hist * nblk
    upw = n_units // NW

    mesh = plsc.VectorSubcoreMesh(core_axis_name="c", subcore_axis_name="s")

    @functools.partial(
        pl.kernel,
        mesh=mesh,
        out_type=jax.ShapeDtypeStruct((hist, D, batch), jnp.float32),
        compiler_params=pltpu.CompilerParams(needs_layout_passes=False),
        scratch_types=[
            pltpu.VMEM((2, BB), jnp.int32),
            pltpu.VMEM((2, BB), jnp.int32),
            pltpu.VMEM((2, BB), jnp.int32),
            pltpu.VMEM((2, BB, 128), jnp.float32),
            pltpu.VMEM((2, D, BB), jnp.float32),
            pltpu.SemaphoreType.DMA,
            pltpu.SemaphoreType.DMA,
            pltpu.SemaphoreType.DMA,
        ],
    )
    def k(idx_hbm, table_hbm, out_hbm, iv, wv, pv, buf, oblk, isem, gsem, wsem):
        wid = lax.axis_index("s") * NC + lax.axis_index("c")
        lane = lax.iota(jnp.int32, 16)
        ubase = wid * upw

        def stage_idx(t, sem_async):
            u = ubase + t
            h = u // nblk
            blk = u % nblk
            src = idx_hbm.at[h, pl.ds(blk * BB, BB)]
            dst = iv.at[t % 2]
            if sem_async:
                pltpu.async_copy(src, dst, isem)
            else:
                pltpu.sync_copy(src, dst)

        def wait_idx():
            pltpu.make_async_copy(idx_hbm.at[0, pl.ds(0, BB)], iv.at[0], isem).wait()

        def compute_widx(t):
            s = t % 2
            for g in range(BB // 16):
                v = iv[s, pl.ds(g * 16, 16)]
                wv[s, pl.ds(g * 16, 16)] = lax.shift_right_logical(v, 1)
                pv[s, pl.ds(g * 16, 16)] = lax.shift_left(lax.bitwise_and(v, 1), 6)

        def fire_gather(t):
            pltpu.async_copy(table_hbm.at[wv.at[t % 2]], buf.at[t % 2], gsem)

        def wait_gather():
            pltpu.make_async_copy(
                table_hbm.at[pl.ds(0, BB)], buf.at[0], gsem).wait()

        def select(t):
            s = t % 2
            for j in range(D):
                crot = lax.bitwise_and(j + lane, D - 1)
                for g in range(BB // 16):
                    rows = g * 16 + lane
                    cvec = pv[s, pl.ds(g * 16, 16)] + crot
                    vals = plsc.load_gather(buf.at[s], [rows, cvec])
                    plsc.store_scatter(oblk.at[s], [crot, rows], vals)

        def fire_write(t):
            u = ubase + t
            h = u // nblk
            blk = u % nblk
            pltpu.async_copy(
                oblk.at[t % 2], out_hbm.at[h, :, pl.ds(blk * BB, BB)], wsem)

        def wait_write():
            pltpu.make_async_copy(
                oblk.at[0], out_hbm.at[0, :, pl.ds(0, BB)], wsem).wait()

        stage_idx(0, False)
        compute_widx(0)
        fire_gather(0)
        stage_idx(1, True)

        def body(t, carry):
            @pl.when(t + 1 < upw)
            def _():
                wait_idx()
                compute_widx(t + 1)
                fire_gather(t + 1)

            @pl.when(t + 2 < upw)
            def _():
                stage_idx(t + 2, True)

            wait_gather()

            @pl.when(t >= 2)
            def _():
                wait_write()

            select(t)
            fire_write(t)
            return carry

        lax.fori_loop(0, upw, body, 0)
        wait_write()
        wait_write()

    return k(idx_t, table_wide)


def kernel(itemseq_input, embedding_weight):
    batch, hist = itemseq_input.shape
    idx_t = itemseq_input.astype(jnp.int32).T
    table_wide = embedding_weight.reshape(-1, 2 * D)
    out_t = _emb_lookup(idx_t, table_wide)
    return out_t.transpose(2, 0, 1)


# speedup vs baseline: 2.4342x; 1.8441x over previous
import functools

import jax
import jax.numpy as jnp
from jax import lax
from jax.experimental import pallas as pl
from jax.experimental.pallas import tpu as pltpu
from jax.experimental.pallas import tpu_sc as plsc

D = 64
CH = 128
K = 4
NC = 2
NS = 16
NW = NC * NS


def _emb_lookup(idx2, table, rows):
    n_chunks = idx2.shape[0]
    cpw = n_chunks // NW
    ng = cpw // K

    mesh = plsc.VectorSubcoreMesh(core_axis_name="c", subcore_axis_name="s")

    @functools.partial(
        pl.kernel,
        mesh=mesh,
        out_type=jax.ShapeDtypeStruct((rows, 2 * D), jnp.float32),
        compiler_params=pltpu.CompilerParams(use_tc_tiling_on_sc=False),
        scratch_types=[
            pltpu.VMEM((cpw, CH), jnp.int32),
            pltpu.VMEM((K, CH, D), jnp.float32),
            pltpu.VMEM((K, CH, D), jnp.float32),
            pltpu.SemaphoreType.DMA,
            pltpu.SemaphoreType.DMA,
            pltpu.SemaphoreType.DMA,
            pltpu.SemaphoreType.DMA,
        ],
    )
    def k(idx_hbm, table_hbm, out_hbm, idx_v, buf_a, buf_b, gs_a, gs_b, ss_a, ss_b):
        wid = lax.axis_index("s") * NC + lax.axis_index("c")
        cbase = wid * cpw
        pltpu.sync_copy(idx_hbm.at[pl.ds(cbase, cpw)], idx_v)

        def fire_gathers(buf, sem, g):
            for b in range(K):
                j = g * K + b
                pltpu.async_copy(table_hbm.at[idx_v.at[j]], buf.at[b], sem)

        def wait_gathers(buf, sem):
            for b in range(K):
                pltpu.make_async_copy(table_hbm.at[pl.ds(0, CH)], buf.at[b], sem).wait()

        def fire_scatters(buf, sem, g):
            for b in range(K):
                j = g * K + b
                pltpu.async_copy(
                    buf.at[b],
                    out_hbm.at[pl.ds((cbase + j) * CH, CH), pl.ds(0, D)], sem)

        def wait_scatters(buf, sem):
            for b in range(K):
                pltpu.make_async_copy(
                    buf.at[b], out_hbm.at[pl.ds(0, CH), pl.ds(0, D)], sem).wait()

        fire_gathers(buf_a, gs_a, 0)
        fire_gathers(buf_b, gs_b, 1)

        def body(t, carry):
            ga = 2 * t
            gb = ga + 1
            wait_gathers(buf_a, gs_a)
            fire_scatters(buf_a, ss_a, ga)
            wait_gathers(buf_b, gs_b)
            fire_scatters(buf_b, ss_b, gb)
            wait_scatters(buf_a, ss_a)
            fire_gathers(buf_a, gs_a, ga + 2)
            wait_scatters(buf_b, ss_b)
            fire_gathers(buf_b, gs_b, gb + 2)
            return carry

        lax.fori_loop(0, ng // 2 - 1, body, 0)

        wait_gathers(buf_a, gs_a)
        fire_scatters(buf_a, ss_a, ng - 2)
        wait_gathers(buf_b, gs_b)
        fire_scatters(buf_b, ss_b, ng - 1)
        wait_scatters(buf_a, ss_a)
        wait_scatters(buf_b, ss_b)

    return k(idx2, table)


def kernel(itemseq_input, embedding_weight):
    batch, hist = itemseq_input.shape
    rows = batch * hist
    idx2 = itemseq_input.astype(jnp.int32).reshape(rows // CH, CH)
    out = _emb_lookup(idx2, embedding_weight.astype(jnp.float32), rows)
    return out.reshape(batch, hist, 2 * D)[:, :, :D]
